# baseline (speedup 1.0000x reference)
import functools

import jax
import jax.numpy as jnp
from jax import lax
from jax.experimental import pallas as pl
from jax.experimental.pallas import tpu as pltpu
from jax.experimental.pallas import tpu_sc as plsc


@functools.cache
def _sc_gather_fn(vocab: int, batch: int, width: int):
    info = plsc.get_sparse_core_info()
    ncores = info.num_cores
    nsub = info.num_subcores
    nworkers = ncores * nsub
    assert batch % (8 * nworkers) == 0 and width % info.num_lanes == 0
    bpw = batch // nworkers
    mesh = plsc.VectorSubcoreMesh(core_axis_name="c", subcore_axis_name="s")

    @functools.partial(
        pl.kernel,
        mesh=mesh,
        out_type=jax.ShapeDtypeStruct((batch, width), jnp.float32),
        scratch_types=[
            pltpu.VMEM((bpw,), jnp.int32),
            pltpu.VMEM((bpw, width), jnp.float32),
            pltpu.SemaphoreType.DMA,
        ],
        compiler_params=pltpu.CompilerParams(use_tc_tiling_on_sc=False),
    )
    def gather(table_hbm, idx_hbm, out_hbm, idx_v, rows_v, sem):
        wid = lax.axis_index("s") * ncores + lax.axis_index("c")
        base = wid * bpw
        pltpu.sync_copy(idx_hbm.at[pl.ds(base, bpw)], idx_v)
        pltpu.async_copy(table_hbm.at[idx_v], rows_v, sem).wait()
        pltpu.sync_copy(rows_v, out_hbm.at[pl.ds(base, bpw)])

    return gather


def _tc_body(protos_ref, emb_ref, w_ref, sim_ref, out_ref):
    width = emb_ref.shape[0]
    protos = protos_ref[...]

    @pl.when(pl.program_id(0) == 0)
    def _():
        dot = jnp.sum(protos[:width, :] * emb_ref[...], axis=0)
        sim_ref[...] = 1.0 - 1.0 / (1.0 + jnp.exp((dot - 384.0) * 0.01))

    out_ref[...] = lax.dot_general(
        w_ref[...], protos, (((1,), (0,)), ((), ())),
        preferred_element_type=jnp.float32,
    )


@functools.cache
def _tc_fn(batch: int, width: int, vocab: int, tile_n: int):
    grid = pl.cdiv(vocab, tile_n)
    return pl.pallas_call(
        _tc_body,
        grid=(grid,),
        in_specs=[
            pl.BlockSpec((width + 1, batch), lambda i: (0, 0)),
            pl.BlockSpec((width, batch), lambda i: (0, 0)),
            pl.BlockSpec((tile_n, width + 1), lambda i: (i, 0)),
        ],
        out_specs=(
            pl.BlockSpec((batch,), lambda i: (0,)),
            pl.BlockSpec((tile_n, batch), lambda i: (i, 0)),
        ),
        out_shape=(
            jax.ShapeDtypeStruct((batch,), jnp.float32),
            jax.ShapeDtypeStruct((vocab, batch), jnp.float32),
        ),
        compiler_params=pltpu.CompilerParams(
            dimension_semantics=("arbitrary",),
            vmem_limit_bytes=100 * 1024 * 1024,
        ),
    )


def kernel(relation_embedding, relation_id, prototypes, W, b):
    batch, width = relation_embedding.shape
    vocab = W.shape[0]
    protos = _sc_gather_fn(vocab, batch, width)(
        prototypes, relation_id.astype(jnp.int32)
    )
    tile_n = 4096
    w_aug = jnp.concatenate([W, b[:, None]], axis=1)
    protos_aug = jnp.concatenate(
        [protos.T, jnp.ones((1, batch), jnp.float32)], axis=0)
    sim, logits_t = _tc_fn(batch, width, vocab, tile_n)(
        protos_aug, relation_embedding.T, w_aug
    )
    return sim, logits_t.T


# speedup vs baseline: 1.2413x; 1.2413x over previous
import functools

import jax
import jax.numpy as jnp
from jax import lax
from jax.experimental import pallas as pl
from jax.experimental.pallas import tpu as pltpu
from jax.experimental.pallas import tpu_sc as plsc


@functools.cache
def _sc_gather_fn(vocab: int, batch: int, width: int):
    info = plsc.get_sparse_core_info()
    ncores = info.num_cores
    nsub = info.num_subcores
    nworkers = ncores * nsub
    assert batch % (8 * nworkers) == 0 and width % info.num_lanes == 0
    bpw = batch // nworkers
    mesh = plsc.VectorSubcoreMesh(core_axis_name="c", subcore_axis_name="s")

    @functools.partial(
        pl.kernel,
        mesh=mesh,
        out_type=jax.ShapeDtypeStruct((batch, width), jnp.float32),
        scratch_types=[
            pltpu.VMEM((bpw,), jnp.int32),
            pltpu.VMEM((bpw, width), jnp.float32),
            pltpu.SemaphoreType.DMA,
        ],
        compiler_params=pltpu.CompilerParams(use_tc_tiling_on_sc=False),
    )
    def gather(table_hbm, idx_hbm, out_hbm, idx_v, rows_v, sem):
        wid = lax.axis_index("s") * ncores + lax.axis_index("c")
        base = wid * bpw
        pltpu.sync_copy(idx_hbm.at[pl.ds(base, bpw)], idx_v)
        pltpu.async_copy(table_hbm.at[idx_v], rows_v, sem).wait()
        pltpu.sync_copy(rows_v, out_hbm.at[pl.ds(base, bpw)])

    return gather


def _tc_body(tile_n, protos_ref, emb_ref, w_ref, b_ref, sim_ref, out_ref):
    protos = protos_ref[...]

    @pl.when(pl.program_id(0) == 0)
    def _():
        dot = jnp.sum(protos * emb_ref[...], axis=0)
        sim_ref[...] = 1.0 - 1.0 / (1.0 + jnp.exp((dot - 384.0) * 0.01))

    acc = lax.dot_general(
        w_ref[...], protos, (((1,), (0,)), ((), ())),
        preferred_element_type=jnp.float32,
    )
    nrow = b_ref.shape[1]
    e_sel = jnp.where(
        lax.broadcasted_iota(jnp.int32, (tile_n, nrow), 0) // 128
        == lax.broadcasted_iota(jnp.int32, (tile_n, nrow), 1),
        1.0, 0.0)
    m1 = lax.dot_general(
        e_sel, b_ref[0], (((1,), (0,)), ((), ())),
        preferred_element_type=jnp.float32,
    )
    lane_pick = (
        lax.broadcasted_iota(jnp.int32, (tile_n, 128), 0) % 128
        == lax.broadcasted_iota(jnp.int32, (tile_n, 128), 1))
    bias_col = jnp.sum(jnp.where(lane_pick, m1, 0.0), axis=1)
    out_ref[...] = acc + bias_col[:, None]


@functools.cache
def _tc_fn(batch: int, width: int, vocab: int, tile_n: int):
    grid = pl.cdiv(vocab, tile_n)
    return pl.pallas_call(
        functools.partial(_tc_body, tile_n),
        grid=(grid,),
        in_specs=[
            pl.BlockSpec((width, batch), lambda i: (0, 0)),
            pl.BlockSpec((width, batch), lambda i: (0, 0)),
            pl.BlockSpec((tile_n, width), lambda i: (i, 0)),
            pl.BlockSpec((1, tile_n // 128, 128), lambda i: (i, 0, 0)),
        ],
        out_specs=(
            pl.BlockSpec((batch,), lambda i: (0,)),
            pl.BlockSpec((tile_n, batch), lambda i: (i, 0)),
        ),
        out_shape=(
            jax.ShapeDtypeStruct((batch,), jnp.float32),
            jax.ShapeDtypeStruct((vocab, batch), jnp.float32),
        ),
        compiler_params=pltpu.CompilerParams(
            dimension_semantics=("arbitrary",),
            vmem_limit_bytes=100 * 1024 * 1024,
        ),
    )


def kernel(relation_embedding, relation_id, prototypes, W, b):
    batch, width = relation_embedding.shape
    vocab = W.shape[0]
    protos = _sc_gather_fn(vocab, batch, width)(
        prototypes, relation_id.astype(jnp.int32)
    )
    tile_n = 4096
    ntiles = pl.cdiv(vocab, tile_n)
    b_pad = jnp.pad(b, (0, ntiles * tile_n - vocab)).reshape(
        ntiles, tile_n // 128, 128)
    sim, logits_t = _tc_fn(batch, width, vocab, tile_n)(
        protos.T, relation_embedding.T, W, b_pad
    )
    return sim, logits_t.T


# speedup vs baseline: 1.2429x; 1.0013x over previous
import functools

import jax
import jax.numpy as jnp
from jax import lax
from jax.experimental import pallas as pl
from jax.experimental.pallas import tpu as pltpu
from jax.experimental.pallas import tpu_sc as plsc


@functools.cache
def _sc_gather_fn(vocab: int, batch: int, width: int):
    info = plsc.get_sparse_core_info()
    ncores = info.num_cores
    nsub = info.num_subcores
    nworkers = ncores * nsub
    assert batch % (8 * nworkers) == 0 and width % info.num_lanes == 0
    bpw = batch // nworkers
    mesh = plsc.VectorSubcoreMesh(core_axis_name="c", subcore_axis_name="s")

    @functools.partial(
        pl.kernel,
        mesh=mesh,
        out_type=jax.ShapeDtypeStruct((batch, width), jnp.float32),
        scratch_types=[
            pltpu.VMEM((bpw,), jnp.int32),
            pltpu.VMEM((bpw, width), jnp.float32),
            pltpu.SemaphoreType.DMA,
        ],
        compiler_params=pltpu.CompilerParams(use_tc_tiling_on_sc=False),
    )
    def gather(table_hbm, idx_hbm, out_hbm, idx_v, rows_v, sem):
        wid = lax.axis_index("s") * ncores + lax.axis_index("c")
        base = wid * bpw
        pltpu.sync_copy(idx_hbm.at[pl.ds(base, bpw)], idx_v)
        pltpu.async_copy(table_hbm.at[idx_v], rows_v, sem).wait()
        pltpu.sync_copy(rows_v, out_hbm.at[pl.ds(base, bpw)])

    return gather


def _tc_body(tile_n, protos_ref, emb_ref, w_ref, b_ref, sim_ref, out_ref,
             esel_vmem, pick_vmem):
    protos = protos_ref[...]
    nrow = b_ref.shape[1]

    @pl.when(pl.program_id(0) == 0)
    def _():
        dot = jnp.sum(protos * emb_ref[...], axis=0)
        sim_ref[...] = 1.0 - 1.0 / (1.0 + jnp.exp((dot - 384.0) * 0.01))
        esel_vmem[...] = jnp.where(
            lax.broadcasted_iota(jnp.int32, (tile_n, nrow), 0) // 128
            == lax.broadcasted_iota(jnp.int32, (tile_n, nrow), 1),
            1.0, 0.0)
        pick_vmem[...] = jnp.where(
            lax.broadcasted_iota(jnp.int32, (tile_n, 128), 0) % 128
            == lax.broadcasted_iota(jnp.int32, (tile_n, 128), 1),
            1.0, 0.0)

    acc = lax.dot_general(
        w_ref[...], protos, (((1,), (0,)), ((), ())),
        preferred_element_type=jnp.float32,
    )
    m1 = lax.dot_general(
        esel_vmem[...], b_ref[0], (((1,), (0,)), ((), ())),
        preferred_element_type=jnp.float32,
    )
    bias_col = jnp.sum(m1 * pick_vmem[...], axis=1)
    out_ref[...] = acc + bias_col[:, None]


@functools.cache
def _tc_fn(batch: int, width: int, vocab: int, tile_n: int):
    grid = pl.cdiv(vocab, tile_n)
    return pl.pallas_call(
        functools.partial(_tc_body, tile_n),
        grid=(grid,),
        in_specs=[
            pl.BlockSpec((width, batch), lambda i: (0, 0)),
            pl.BlockSpec((width, batch), lambda i: (0, 0)),
            pl.BlockSpec((tile_n, width), lambda i: (i, 0)),
            pl.BlockSpec((1, tile_n // 128, 128), lambda i: (i, 0, 0)),
        ],
        out_specs=(
            pl.BlockSpec((batch,), lambda i: (0,)),
            pl.BlockSpec((tile_n, batch), lambda i: (i, 0)),
        ),
        out_shape=(
            jax.ShapeDtypeStruct((batch,), jnp.float32),
            jax.ShapeDtypeStruct((vocab, batch), jnp.float32),
        ),
        scratch_shapes=[
            pltpu.VMEM((tile_n, tile_n // 128), jnp.float32),
            pltpu.VMEM((tile_n, 128), jnp.float32),
        ],
        compiler_params=pltpu.CompilerParams(
            dimension_semantics=("arbitrary",),
            vmem_limit_bytes=100 * 1024 * 1024,
        ),
    )


def kernel(relation_embedding, relation_id, prototypes, W, b):
    batch, width = relation_embedding.shape
    vocab = W.shape[0]
    protos = _sc_gather_fn(vocab, batch, width)(
        prototypes, relation_id.astype(jnp.int32)
    )
    tile_n = 4096
    ntiles = pl.cdiv(vocab, tile_n)
    b_pad = jnp.pad(b, (0, ntiles * tile_n - vocab)).reshape(
        ntiles, tile_n // 128, 128)
    sim, logits_t = _tc_fn(batch, width, vocab, tile_n)(
        protos.T, relation_embedding.T, W, b_pad
    )
    return sim, logits_t.T


# speedup vs baseline: 1.3872x; 1.1161x over previous
import functools

import jax
import jax.numpy as jnp
from jax import lax
from jax.experimental import pallas as pl
from jax.experimental.pallas import tpu as pltpu
from jax.experimental.pallas import tpu_sc as plsc


@functools.cache
def _sc_gather_fn(vocab: int, batch: int, width: int):
    info = plsc.get_sparse_core_info()
    ncores = info.num_cores
    nsub = info.num_subcores
    nworkers = ncores * nsub
    assert batch % (8 * nworkers) == 0 and width % info.num_lanes == 0
    bpw = batch // nworkers
    mesh = plsc.VectorSubcoreMesh(core_axis_name="c", subcore_axis_name="s")

    @functools.partial(
        pl.kernel,
        mesh=mesh,
        out_type=jax.ShapeDtypeStruct((batch, width), jnp.float32),
        scratch_types=[
            pltpu.VMEM((bpw,), jnp.int32),
            pltpu.VMEM((bpw, width), jnp.float32),
            pltpu.SemaphoreType.DMA,
        ],
        compiler_params=pltpu.CompilerParams(use_tc_tiling_on_sc=False),
    )
    def gather(table_hbm, idx_hbm, out_hbm, idx_v, rows_v, sem):
        wid = lax.axis_index("s") * ncores + lax.axis_index("c")
        base = wid * bpw
        pltpu.sync_copy(idx_hbm.at[pl.ds(base, bpw)], idx_v)
        pltpu.async_copy(table_hbm.at[idx_v], rows_v, sem).wait()
        pltpu.sync_copy(rows_v, out_hbm.at[pl.ds(base, bpw)])

    return gather


def _tc_body(protos_ref, emb_ref, w_ref, sim_ref, out_ref):
    width = emb_ref.shape[0]
    protos = protos_ref[...]

    @pl.when(pl.program_id(0) == 0)
    def _():
        dot = jnp.sum(protos[:width, :] * emb_ref[...], axis=0)
        sim_ref[...] = 1.0 - 1.0 / (1.0 + jnp.exp((dot - 384.0) * 0.01))

    out_ref[...] = lax.dot_general(
        w_ref[...], protos, (((0,), (0,)), ((), ())),
        preferred_element_type=jnp.float32,
    )


@functools.cache
def _tc_fn(batch: int, width: int, vocab: int, tile_n: int):
    grid = pl.cdiv(vocab, tile_n)
    return pl.pallas_call(
        _tc_body,
        grid=(grid,),
        in_specs=[
            pl.BlockSpec((width + 1, batch), lambda i: (0, 0)),
            pl.BlockSpec((width, batch), lambda i: (0, 0)),
            pl.BlockSpec((width + 1, tile_n), lambda i: (0, i)),
        ],
        out_specs=(
            pl.BlockSpec((batch,), lambda i: (0,)),
            pl.BlockSpec((tile_n, batch), lambda i: (i, 0)),
        ),
        out_shape=(
            jax.ShapeDtypeStruct((batch,), jnp.float32),
            jax.ShapeDtypeStruct((vocab, batch), jnp.float32),
        ),
        compiler_params=pltpu.CompilerParams(
            dimension_semantics=("arbitrary",),
            vmem_limit_bytes=100 * 1024 * 1024,
        ),
    )


def kernel(relation_embedding, relation_id, prototypes, W, b):
    batch, width = relation_embedding.shape
    vocab = W.shape[0]
    protos = _sc_gather_fn(vocab, batch, width)(
        prototypes, relation_id.astype(jnp.int32)
    )
    tile_n = 4096
    w_aug = jnp.concatenate([W.T, b[None, :]], axis=0)
    protos_aug = jnp.concatenate(
        [protos.T, jnp.ones((1, batch), jnp.float32)], axis=0)
    sim, logits_t = _tc_fn(batch, width, vocab, tile_n)(
        protos_aug, relation_embedding.T, w_aug
    )
    return sim, logits_t.T
